# Initial kernel scaffold; baseline (speedup 1.0000x reference)
import jax
import jax.numpy as jnp
from jax.experimental import pallas as pl


def kernel(lap_indices, lap_values, x, weight, bias):
    raise NotImplementedError("write your pallas kernel here")


# speedup vs baseline: 6.3715x; 6.3715x over previous
import functools

import jax
import jax.numpy as jnp
from jax import lax
from jax.experimental import pallas as pl
from jax.experimental.pallas import tpu as pltpu
from jax.experimental.pallas import tpu_sc as plsc

V = 10000
VP = 10240
F = 128
E = 320000
KORD = 4
NC = 2
NS = 16
NW = NC * NS
L = 16
W = 128
NBLK = 2560
EPAD = NBLK * W
BLKS_PER_TILE = NBLK // NW
GROUPS = BLKS_PER_TILE // 8
RPT = VP // NS

_mesh = plsc.VectorSubcoreMesh(core_axis_name="c", subcore_axis_name="s")


_SPMM_SCRATCH = [
    pltpu.VMEM((8, W), jnp.int32),
    pltpu.VMEM((8, W), jnp.int32),
    pltpu.VMEM((8 * W,), jnp.float32),
    pltpu.VMEM((W, F), jnp.float32),
    pltpu.VMEM_SHARED((VP, F), jnp.float32),
    pltpu.SemaphoreType.DMA,
]


def _spmm_body(cols, rows, vals, xin, out, colw, roww, valw, rowbuf, acc, sem):
    cid = lax.axis_index("c")
    sid = lax.axis_index("s")
    wid = cid * NS + sid

    def _zero_rowbuf(e, _):
        z = jnp.zeros((L,), jnp.float32)
        for q in range(F // L):
            rowbuf[e, pl.ds(q * L, L)] = z
        return 0

    lax.fori_loop(0, W, _zero_rowbuf, 0)
    r0 = sid * RPT
    for k in range(RPT // W):
        pltpu.sync_copy(rowbuf, acc.at[pl.ds(r0 + k * W, W)])
    plsc.subcore_barrier()

    blk0 = wid * BLKS_PER_TILE

    def _window(j):
        pltpu.async_copy(xin.at[colw.at[j]], rowbuf, sem).wait()

        def _scale16(g, _):
            grp = valw[pl.ds(j * W + g * L, L)]
            for u in range(L):
                e = g * L + u
                v = grp[u]
                for q in range(F // L):
                    rowbuf[e, pl.ds(q * L, L)] = rowbuf[e, pl.ds(q * L, L)] * v
            return 0

        lax.fori_loop(0, W // L, _scale16, 0)

        pltpu.sync_copy(rowbuf, acc.at[roww.at[j]], add=True)

    def _outer(g, _):
        b = blk0 + g * 8
        pltpu.sync_copy(cols.at[pl.ds(b, 8)], colw)
        pltpu.sync_copy(rows.at[pl.ds(b, 8)], roww)
        pltpu.sync_copy(vals.at[pl.ds(b * W, 8 * W)], valw)

        def _win_wrap(j, _):
            _window(j)
            return 0

        lax.fori_loop(0, 8, _win_wrap, 0)
        return 0

    lax.fori_loop(0, GROUPS, _outer, 0)

    plsc.subcore_barrier()

    for k in range(RPT // W):
        pltpu.sync_copy(acc.at[pl.ds(r0 + k * W, W)],
                        out.at[cid, pl.ds(r0 + k * W, W)])


_spmm_sc = pl.kernel(
    _spmm_body,
    out_type=jax.ShapeDtypeStruct((NC, VP, F), jnp.float32),
    mesh=_mesh,
    scratch_types=_SPMM_SCRATCH,
)


def _combine_first_body(p_ref, o_ref):
    o_ref[...] = p_ref[0] + p_ref[1]


def _combine_rec_body(p_ref, xprev_ref, o_ref):
    o_ref[...] = 2.0 * (p_ref[0] + p_ref[1]) - xprev_ref[...]


def _combine_first(p):
    return pl.pallas_call(
        _combine_first_body,
        out_shape=jax.ShapeDtypeStruct((VP, F), jnp.float32),
    )(p)


def _combine_rec(p, xprev):
    return pl.pallas_call(
        _combine_rec_body,
        out_shape=jax.ShapeDtypeStruct((VP, F), jnp.float32),
    )(p, xprev)


_MM_ROWS = 2048


def _matmul_body(x0_ref, x1_ref, x2_ref, x3_ref, w_ref, b_ref, o_ref):
    acc = jnp.dot(x0_ref[...], w_ref[0], preferred_element_type=jnp.float32)
    acc += jnp.dot(x1_ref[...], w_ref[1], preferred_element_type=jnp.float32)
    acc += jnp.dot(x2_ref[...], w_ref[2], preferred_element_type=jnp.float32)
    acc += jnp.dot(x3_ref[...], w_ref[3], preferred_element_type=jnp.float32)
    o_ref[...] = acc + b_ref[...]


def _matmul(x0, x1, x2, x3, weight, bias2d):
    xspec = pl.BlockSpec((_MM_ROWS, F), lambda i: (i, 0))
    return pl.pallas_call(
        _matmul_body,
        grid=(VP // _MM_ROWS,),
        in_specs=[xspec, xspec, xspec, xspec,
                  pl.BlockSpec((KORD, F, F), lambda i: (0, 0, 0)),
                  pl.BlockSpec((1, F), lambda i: (0, 0))],
        out_specs=pl.BlockSpec((_MM_ROWS, F), lambda i: (i, 0)),
        out_shape=jax.ShapeDtypeStruct((VP, F), jnp.float32),
    )(x0, x1, x2, x3, weight, bias2d)


def kernel(lap_indices, lap_values, x, weight, bias):
    npad = EPAD - E
    spread = (jnp.arange(npad, dtype=jnp.int32) * 13) % V
    rows = jnp.concatenate(
        [lap_indices[0].astype(jnp.int32), spread]).reshape(NBLK, W)
    cols = jnp.concatenate(
        [lap_indices[1].astype(jnp.int32), spread]).reshape(NBLK, W)
    vals = jnp.concatenate(
        [lap_values.astype(jnp.float32), jnp.zeros((npad,), jnp.float32)])
    x0 = jnp.pad(x.reshape(V, F), ((0, VP - V), (0, 0)))
    bias2d = bias.reshape(1, F)
    weff = jnp.transpose(
        weight.reshape(KORD * F, F).reshape(F, KORD, F), (1, 0, 2))

    p = _spmm_sc(cols, rows, vals, x0)
    x1 = _combine_first(p)
    p = _spmm_sc(cols, rows, vals, x1)
    x2 = _combine_rec(p, x0)
    p = _spmm_sc(cols, rows, vals, x2)
    x3 = _combine_rec(p, x1)
    out = _matmul(x0, x1, x2, x3, weff, bias2d)
    return out[:V].reshape(1, V, F)


# speedup vs baseline: 9.7658x; 1.5327x over previous
import jax
import jax.numpy as jnp
from jax import lax
from jax.experimental import pallas as pl
from jax.experimental.pallas import tpu as pltpu
from jax.experimental.pallas import tpu_sc as plsc

V = 10000
VP = 10240
F = 128
E = 320000
KORD = 4
NC = 2
NS = 16
NW = NC * NS
L = 16
W = 128
SEG = 64
NBLK = 2560
EPAD = NBLK * W
NSEG_TOTAL = EPAD // SEG
WPT = NBLK // NW
SPT = WPT * 2
SEG_CHUNK = 32
PAIR_CHUNK = SEG_CHUNK // 2
NCHUNK = SPT // SEG_CHUNK
RPT = VP // NS

_mesh = plsc.VectorSubcoreMesh(core_axis_name="c", subcore_axis_name="s")

_SPMM_SCRATCH = [
    pltpu.VMEM((SEG_CHUNK, SEG), jnp.int32),
    pltpu.VMEM((PAIR_CHUNK, W), jnp.int32),
    pltpu.VMEM((SEG_CHUNK * SEG,), jnp.float32),
    pltpu.VMEM((4 * SEG, F), jnp.float32),
    pltpu.VMEM_SHARED((VP, F), jnp.float32),
    pltpu.SemaphoreType.DMA,
    pltpu.SemaphoreType.DMA,
    pltpu.SemaphoreType.DMA,
    pltpu.SemaphoreType.DMA,
    pltpu.SemaphoreType.DMA,
    pltpu.SemaphoreType.DMA,
]


def _spmm_body(cols, rows, vals, xin, out,
               colw, roww, valw, ring, acc, gs0, gs1, gs2, gs3, ps0, ps1):
    gsems = (gs0, gs1, gs2, gs3)
    psems = (ps0, ps1)
    cid = lax.axis_index("c")
    sid = lax.axis_index("s")
    wid = cid * NS + sid

    def _zero_rowbuf(e, _):
        z = jnp.zeros((L,), jnp.float32)
        for q in range(F // L):
            ring[e, pl.ds(q * L, L)] = z
        return 0

    lax.fori_loop(0, W, _zero_rowbuf, 0)
    r0 = sid * RPT
    for k in range(RPT // W):
        pltpu.sync_copy(ring.at[pl.ds(0, W)], acc.at[pl.ds(r0 + k * W, W)])
    plsc.subcore_barrier()

    def _stage(c):
        pltpu.sync_copy(cols.at[pl.ds(wid * SPT + c * SEG_CHUNK, SEG_CHUNK)],
                        colw)
        pltpu.sync_copy(rows.at[pl.ds(wid * WPT + c * PAIR_CHUNK, PAIR_CHUNK)],
                        roww)
        pltpu.sync_copy(
            vals.at[pl.ds((wid * WPT + c * PAIR_CHUNK) * W, SEG_CHUNK * SEG)],
            valw)

    def _gather(ls, k):
        pltpu.async_copy(xin.at[colw.at[ls]],
                         ring.at[pl.ds(k * SEG, SEG)], gsems[k])

    def _gwait(ls, k):
        pltpu.make_async_copy(xin.at[colw.at[ls]],
                              ring.at[pl.ds(k * SEG, SEG)], gsems[k]).wait()

    def _scale(ls, k):
        base = k * SEG

        def _s16(g, _):
            grp = valw[pl.ds(ls * SEG + g * L, L)]
            for u in range(L):
                e = base + g * L + u
                v = grp[u]
                for q in range(F // L):
                    ring[e, pl.ds(q * L, L)] = ring[e, pl.ds(q * L, L)] * v
            return 0

        lax.fori_loop(0, SEG // L, _s16, 0)

    def _scatter_pair(lp, half, sem):
        pltpu.async_copy(ring.at[pl.ds(half * W, W)],
                         acc.at[roww.at[lp]], sem, add=True)

    def _pwait(half, sem):
        pltpu.make_async_copy(ring.at[pl.ds(half * W, W)],
                              acc.at[roww.at[0]], sem).wait()

    def _iter(i):
        for k in range(4):
            ls = 4 * i + k
            _gwait(ls, k)
            _scale(ls, k)
            if k == 1:
                _scatter_pair(2 * i, 0, psems[0])
            if k == 3:
                _scatter_pair(2 * i + 1, 1, psems[1])
            if k == 0:
                @pl.when(i > 0)
                def _():
                    _pwait(1, psems[1])

                _gather(ls + 2, 2)
            elif k == 1:
                _gather(ls + 2, 3)
            elif k == 2:
                _pwait(0, psems[0])
                _gather(ls + 2, 0)
            else:
                _gather(ls + 2, 1)

    _stage(0)
    _gather(0, 0)
    _gather(1, 1)

    def _body(i, _):
        _iter(i)
        return 0

    def _chunk(c, _):
        lax.fori_loop(0, SEG_CHUNK // 4 - 1, _body, 0)

        i = SEG_CHUNK // 4 - 1
        for k in range(4):
            ls = 4 * i + k
            _gwait(ls, k)
            _scale(ls, k)
            if k == 0:
                _pwait(1, psems[1])
                _gather(ls + 2, 2)
            elif k == 1:
                _scatter_pair(2 * i, 0, psems[0])
                _gather(ls + 2, 3)
            elif k == 3:
                _scatter_pair(2 * i + 1, 1, psems[1])

        _pwait(0, psems[0])
        _pwait(1, psems[1])

        @pl.when(c + 1 < NCHUNK)
        def _():
            _stage(c + 1)
            _gather(0, 0)
            _gather(1, 1)

        return 0

    lax.fori_loop(0, NCHUNK, _chunk, 0)

    plsc.subcore_barrier()

    for k in range(RPT // W):
        pltpu.sync_copy(acc.at[pl.ds(r0 + k * W, W)],
                        out.at[cid, pl.ds(r0 + k * W, W)])


_spmm_sc = pl.kernel(
    _spmm_body,
    out_type=jax.ShapeDtypeStruct((NC, VP, F), jnp.float32),
    mesh=_mesh,
    scratch_types=_SPMM_SCRATCH,
)


def _combine_first_body(p_ref, o_ref):
    o_ref[...] = p_ref[0] + p_ref[1]


def _combine_rec_body(p_ref, xprev_ref, o_ref):
    o_ref[...] = 2.0 * (p_ref[0] + p_ref[1]) - xprev_ref[...]


def _combine_first(p):
    return pl.pallas_call(
        _combine_first_body,
        out_shape=jax.ShapeDtypeStruct((VP, F), jnp.float32),
    )(p)


def _combine_rec(p, xprev):
    return pl.pallas_call(
        _combine_rec_body,
        out_shape=jax.ShapeDtypeStruct((VP, F), jnp.float32),
    )(p, xprev)


_MM_ROWS = 2048


def _matmul_body(x0_ref, x1_ref, x2_ref, x3_ref, w_ref, b_ref, o_ref):
    acc = jnp.dot(x0_ref[...], w_ref[0], preferred_element_type=jnp.float32)
    acc += jnp.dot(x1_ref[...], w_ref[1], preferred_element_type=jnp.float32)
    acc += jnp.dot(x2_ref[...], w_ref[2], preferred_element_type=jnp.float32)
    acc += jnp.dot(x3_ref[...], w_ref[3], preferred_element_type=jnp.float32)
    o_ref[...] = acc + b_ref[...]


def _matmul(x0, x1, x2, x3, weight, bias2d):
    xspec = pl.BlockSpec((_MM_ROWS, F), lambda i: (i, 0))
    return pl.pallas_call(
        _matmul_body,
        grid=(VP // _MM_ROWS,),
        in_specs=[xspec, xspec, xspec, xspec,
                  pl.BlockSpec((KORD, F, F), lambda i: (0, 0, 0)),
                  pl.BlockSpec((1, F), lambda i: (0, 0))],
        out_specs=pl.BlockSpec((_MM_ROWS, F), lambda i: (i, 0)),
        out_shape=jax.ShapeDtypeStruct((VP, F), jnp.float32),
    )(x0, x1, x2, x3, weight, bias2d)


def kernel(lap_indices, lap_values, x, weight, bias):
    npad = EPAD - E
    spread = (jnp.arange(npad, dtype=jnp.int32) * 13) % V
    rows = jnp.concatenate(
        [lap_indices[0].astype(jnp.int32), spread]).reshape(NBLK, W)
    cols = jnp.concatenate(
        [lap_indices[1].astype(jnp.int32), spread]).reshape(NSEG_TOTAL, SEG)
    vals = jnp.concatenate(
        [lap_values.astype(jnp.float32), jnp.zeros((npad,), jnp.float32)])
    x0 = jnp.pad(x.reshape(V, F), ((0, VP - V), (0, 0)))
    bias2d = bias.reshape(1, F)
    weff = jnp.transpose(
        weight.reshape(KORD * F, F).reshape(F, KORD, F), (1, 0, 2))

    p = _spmm_sc(cols, rows, vals, x0)
    x1 = _combine_first(p)
    p = _spmm_sc(cols, rows, vals, x1)
    x2 = _combine_rec(p, x0)
    p = _spmm_sc(cols, rows, vals, x2)
    x3 = _combine_rec(p, x1)
    out = _matmul(x0, x1, x2, x3, weff, bias2d)
    return out[:V].reshape(1, V, F)


# speedup vs baseline: 9.8633x; 1.0100x over previous
import jax
import jax.numpy as jnp
from jax import lax
from jax.experimental import pallas as pl
from jax.experimental.pallas import tpu as pltpu
from jax.experimental.pallas import tpu_sc as plsc

V = 10000
VP = 10240
F = 128
E = 320000
KORD = 4
NC = 2
NS = 16
NW = NC * NS
L = 16
W = 128
SEG = 64
NBLK = 2560
EPAD = NBLK * W
NSEG_TOTAL = EPAD // SEG
WPT = NBLK // NW
SPT = WPT * 2
SEG_CHUNK = 32
PAIR_CHUNK = SEG_CHUNK // 2
NCHUNK = SPT // SEG_CHUNK
RPT = VP // NS

_mesh = plsc.VectorSubcoreMesh(core_axis_name="c", subcore_axis_name="s")

_SPMM_SCRATCH = [
    pltpu.VMEM((SEG_CHUNK, SEG), jnp.int32),
    pltpu.VMEM((SEG_CHUNK, SEG), jnp.int32),
    pltpu.VMEM((SEG_CHUNK * SEG,), jnp.float32),
    pltpu.VMEM((4 * SEG, F), jnp.float32),
    pltpu.VMEM_SHARED((VP, F), jnp.float32),
    pltpu.SemaphoreType.DMA,
    pltpu.SemaphoreType.DMA,
    pltpu.SemaphoreType.DMA,
    pltpu.SemaphoreType.DMA,
    pltpu.SemaphoreType.DMA,
    pltpu.SemaphoreType.DMA,
    pltpu.SemaphoreType.DMA,
    pltpu.SemaphoreType.DMA,
]


def _spmm_body(cols, rows, vals, xin, out,
               colw, roww, valw, ring, acc,
               gs0, gs1, gs2, gs3, ss0, ss1, ss2, ss3):
    gsems = (gs0, gs1, gs2, gs3)
    ssems = (ss0, ss1, ss2, ss3)
    cid = lax.axis_index("c")
    sid = lax.axis_index("s")
    wid = cid * NS + sid

    def _zero_rowbuf(e, _):
        z = jnp.zeros((L,), jnp.float32)
        for q in range(F // L):
            ring[e, pl.ds(q * L, L)] = z
        return 0

    lax.fori_loop(0, W, _zero_rowbuf, 0)
    r0 = sid * RPT
    for k in range(RPT // W):
        pltpu.sync_copy(ring.at[pl.ds(0, W)], acc.at[pl.ds(r0 + k * W, W)])
    plsc.subcore_barrier()

    def _stage(c):
        pltpu.sync_copy(cols.at[pl.ds(wid * SPT + c * SEG_CHUNK, SEG_CHUNK)],
                        colw)
        pltpu.sync_copy(rows.at[pl.ds(wid * SPT + c * SEG_CHUNK, SEG_CHUNK)],
                        roww)
        pltpu.sync_copy(
            vals.at[pl.ds((wid * SPT + c * SEG_CHUNK) * SEG, SEG_CHUNK * SEG)],
            valw)

    def _gather(ls, k):
        pltpu.async_copy(xin.at[colw.at[ls]],
                         ring.at[pl.ds(k * SEG, SEG)], gsems[k])

    def _gwait(ls, k):
        pltpu.make_async_copy(xin.at[colw.at[ls]],
                              ring.at[pl.ds(k * SEG, SEG)], gsems[k]).wait()

    def _scale(ls, k):
        base = k * SEG

        def _s16(g, _):
            grp = valw[pl.ds(ls * SEG + g * L, L)]
            for u in range(L):
                e = base + g * L + u
                v = grp[u]
                for q in range(F // L):
                    ring[e, pl.ds(q * L, L)] = ring[e, pl.ds(q * L, L)] * v
            return 0

        lax.fori_loop(0, SEG // L, _s16, 0)

    def _scatter(ls, k):
        pltpu.async_copy(ring.at[pl.ds(k * SEG, SEG)],
                         acc.at[roww.at[ls]], ssems[k], add=True)

    def _swait(k):
        pltpu.make_async_copy(ring.at[pl.ds(k * SEG, SEG)],
                              acc.at[roww.at[0]], ssems[k]).wait()

    def _iter(i):
        for k in range(4):
            ls = 4 * i + k
            _gwait(ls, k)
            _scale(ls, k)
            _scatter(ls, k)
            j = (k + 2) % 4
            if k < 2:
                @pl.when(i > 0)
                def _():
                    _swait(j)

                _gather(ls + 2, j)
            else:
                _swait(j)
                _gather(ls + 2, j)

    _stage(0)
    _gather(0, 0)
    _gather(1, 1)

    def _body(i, _):
        _iter(i)
        return 0

    def _chunk(c, _):
        lax.fori_loop(0, SEG_CHUNK // 4 - 1, _body, 0)

        i = SEG_CHUNK // 4 - 1
        for k in range(4):
            ls = 4 * i + k
            _gwait(ls, k)
            _scale(ls, k)
            _scatter(ls, k)
            if k < 2:
                _swait((k + 2) % 4)
                _gather(ls + 2, (k + 2) % 4)

        for k in range(4):
            _swait(k)

        @pl.when(c + 1 < NCHUNK)
        def _():
            _stage(c + 1)
            _gather(0, 0)
            _gather(1, 1)

        return 0

    lax.fori_loop(0, NCHUNK, _chunk, 0)

    plsc.subcore_barrier()

    for k in range(RPT // W):
        pltpu.sync_copy(acc.at[pl.ds(r0 + k * W, W)],
                        out.at[cid, pl.ds(r0 + k * W, W)])


_spmm_sc = pl.kernel(
    _spmm_body,
    out_type=jax.ShapeDtypeStruct((NC, VP, F), jnp.float32),
    mesh=_mesh,
    scratch_types=_SPMM_SCRATCH,
)


def _combine_first_body(p_ref, o_ref):
    o_ref[...] = p_ref[0] + p_ref[1]


def _combine_rec_body(p_ref, xprev_ref, o_ref):
    o_ref[...] = 2.0 * (p_ref[0] + p_ref[1]) - xprev_ref[...]


def _combine_first(p):
    return pl.pallas_call(
        _combine_first_body,
        out_shape=jax.ShapeDtypeStruct((VP, F), jnp.float32),
    )(p)


def _combine_rec(p, xprev):
    return pl.pallas_call(
        _combine_rec_body,
        out_shape=jax.ShapeDtypeStruct((VP, F), jnp.float32),
    )(p, xprev)


_MM_ROWS = 2048


def _matmul_body(x0_ref, x1_ref, x2_ref, x3_ref, w_ref, b_ref, o_ref):
    acc = jnp.dot(x0_ref[...], w_ref[0], preferred_element_type=jnp.float32)
    acc += jnp.dot(x1_ref[...], w_ref[1], preferred_element_type=jnp.float32)
    acc += jnp.dot(x2_ref[...], w_ref[2], preferred_element_type=jnp.float32)
    acc += jnp.dot(x3_ref[...], w_ref[3], preferred_element_type=jnp.float32)
    o_ref[...] = acc + b_ref[...]


def _matmul(x0, x1, x2, x3, weight, bias2d):
    xspec = pl.BlockSpec((_MM_ROWS, F), lambda i: (i, 0))
    return pl.pallas_call(
        _matmul_body,
        grid=(VP // _MM_ROWS,),
        in_specs=[xspec, xspec, xspec, xspec,
                  pl.BlockSpec((KORD, F, F), lambda i: (0, 0, 0)),
                  pl.BlockSpec((1, F), lambda i: (0, 0))],
        out_specs=pl.BlockSpec((_MM_ROWS, F), lambda i: (i, 0)),
        out_shape=jax.ShapeDtypeStruct((VP, F), jnp.float32),
    )(x0, x1, x2, x3, weight, bias2d)


def kernel(lap_indices, lap_values, x, weight, bias):
    npad = EPAD - E
    spread = (jnp.arange(npad, dtype=jnp.int32) * 13) % V
    rows = jnp.concatenate(
        [lap_indices[0].astype(jnp.int32), spread]).reshape(NSEG_TOTAL, SEG)
    cols = jnp.concatenate(
        [lap_indices[1].astype(jnp.int32), spread]).reshape(NSEG_TOTAL, SEG)
    vals = jnp.concatenate(
        [lap_values.astype(jnp.float32), jnp.zeros((npad,), jnp.float32)])
    x0 = jnp.pad(x.reshape(V, F), ((0, VP - V), (0, 0)))
    bias2d = bias.reshape(1, F)
    weff = jnp.transpose(
        weight.reshape(KORD * F, F).reshape(F, KORD, F), (1, 0, 2))

    p = _spmm_sc(cols, rows, vals, x0)
    x1 = _combine_first(p)
    p = _spmm_sc(cols, rows, vals, x1)
    x2 = _combine_rec(p, x0)
    p = _spmm_sc(cols, rows, vals, x2)
    x3 = _combine_rec(p, x1)
    out = _matmul(x0, x1, x2, x3, weff, bias2d)
    return out[:V].reshape(1, V, F)


# speedup vs baseline: 10.0635x; 1.0203x over previous
import jax
import jax.numpy as jnp
from jax import lax
from jax.experimental import pallas as pl
from jax.experimental.pallas import tpu as pltpu
from jax.experimental.pallas import tpu_sc as plsc

V = 10000
VP = 10240
F = 128
E = 320000
KORD = 4
NC = 2
NS = 16
NW = NC * NS
L = 16
W = 128
SEG = 64
NBLK = 2560
EPAD = NBLK * W
NSEG_TOTAL = EPAD // SEG
WPT = NBLK // NW
SPT = WPT * 2
SEG_CHUNK = 32
PAIR_CHUNK = SEG_CHUNK // 2
NCHUNK = SPT // SEG_CHUNK
RPT = VP // NS

_mesh = plsc.VectorSubcoreMesh(core_axis_name="c", subcore_axis_name="s")

_SPMM_SCRATCH = [
    pltpu.VMEM((SEG_CHUNK, SEG), jnp.int32),
    pltpu.VMEM((SEG_CHUNK, SEG), jnp.int32),
    pltpu.VMEM((SEG_CHUNK * SEG,), jnp.float32),
    pltpu.VMEM((4 * SEG, F), jnp.float32),
    pltpu.VMEM_SHARED((VP, F), jnp.float32),
    pltpu.SemaphoreType.DMA,
    pltpu.SemaphoreType.DMA,
    pltpu.SemaphoreType.DMA,
    pltpu.SemaphoreType.DMA,
    pltpu.SemaphoreType.DMA,
    pltpu.SemaphoreType.DMA,
    pltpu.SemaphoreType.DMA,
    pltpu.SemaphoreType.DMA,
]


def _spmm_body(cols, rows, vals, xin, out,
               colw, roww, valw, ring, acc,
               gs0, gs1, gs2, gs3, ss0, ss1, ss2, ss3):
    gsems = (gs0, gs1, gs2, gs3)
    ssems = (ss0, ss1, ss2, ss3)
    cid = lax.axis_index("c")
    sid = lax.axis_index("s")
    wid = cid * NS + sid

    r0 = sid * RPT

    def _stage(c):
        pltpu.sync_copy(cols.at[pl.ds(wid * SPT + c * SEG_CHUNK, SEG_CHUNK)],
                        colw)
        pltpu.sync_copy(rows.at[pl.ds(wid * SPT + c * SEG_CHUNK, SEG_CHUNK)],
                        roww)
        pltpu.sync_copy(
            vals.at[pl.ds((wid * SPT + c * SEG_CHUNK) * SEG, SEG_CHUNK * SEG)],
            valw)

    def _gather(ls, k):
        pltpu.async_copy(xin.at[colw.at[ls]],
                         ring.at[pl.ds(k * SEG, SEG)], gsems[k])

    def _gwait(ls, k):
        pltpu.make_async_copy(xin.at[colw.at[ls]],
                              ring.at[pl.ds(k * SEG, SEG)], gsems[k]).wait()

    def _scale(ls, k):
        base = k * SEG

        def _s16(g, _):
            grp = valw[pl.ds(ls * SEG + g * L, L)]
            for u in range(L):
                e = base + g * L + u
                v = grp[u]
                for q in range(F // L):
                    ring[e, pl.ds(q * L, L)] = ring[e, pl.ds(q * L, L)] * v
            return 0

        lax.fori_loop(0, SEG // L, _s16, 0)

    def _scatter(ls, k):
        pltpu.async_copy(ring.at[pl.ds(k * SEG, SEG)],
                         acc.at[roww.at[ls]], ssems[k], add=True)

    def _swait(k):
        pltpu.make_async_copy(ring.at[pl.ds(k * SEG, SEG)],
                              acc.at[roww.at[0]], ssems[k]).wait()

    def _iter(i):
        for k in range(4):
            ls = 4 * i + k
            _gwait(ls, k)
            _scale(ls, k)
            _scatter(ls, k)
            j = (k + 2) % 4
            if k < 2:
                @pl.when(i > 0)
                def _():
                    _swait(j)

                _gather(ls + 2, j)
            else:
                _swait(j)
                _gather(ls + 2, j)

    _stage(0)
    _gather(0, 0)
    _gather(1, 1)

    def _zero_rowbuf(e, _):
        z = jnp.zeros((L,), jnp.float32)
        for q in range(F // L):
            ring[W + e, pl.ds(q * L, L)] = z
        return 0

    lax.fori_loop(0, W, _zero_rowbuf, 0)
    for k in range(RPT // W):
        pltpu.async_copy(ring.at[pl.ds(W, W)], acc.at[pl.ds(r0 + k * W, W)],
                         ss0)
    for k in range(RPT // W):
        pltpu.make_async_copy(ring.at[pl.ds(W, W)],
                              acc.at[pl.ds(r0 + k * W, W)], ss0).wait()
    plsc.subcore_barrier()

    def _body(i, _):
        _iter(i)
        return 0

    def _chunk(c, _):
        lax.fori_loop(0, SEG_CHUNK // 4 - 1, _body, 0)

        i = SEG_CHUNK // 4 - 1
        for k in range(4):
            ls = 4 * i + k
            _gwait(ls, k)
            _scale(ls, k)
            _scatter(ls, k)
            if k < 2:
                _swait((k + 2) % 4)
                _gather(ls + 2, (k + 2) % 4)

        for k in range(4):
            _swait(k)

        @pl.when(c + 1 < NCHUNK)
        def _():
            _stage(c + 1)
            _gather(0, 0)
            _gather(1, 1)

        return 0

    lax.fori_loop(0, NCHUNK, _chunk, 0)

    plsc.subcore_barrier()

    for k in range(RPT // W):
        pltpu.async_copy(acc.at[pl.ds(r0 + k * W, W)],
                         out.at[cid, pl.ds(r0 + k * W, W)], ss0)
    for k in range(RPT // W):
        pltpu.make_async_copy(acc.at[pl.ds(r0 + k * W, W)],
                              out.at[cid, pl.ds(r0 + k * W, W)], ss0).wait()


_spmm_sc = pl.kernel(
    _spmm_body,
    out_type=jax.ShapeDtypeStruct((NC, VP, F), jnp.float32),
    mesh=_mesh,
    scratch_types=_SPMM_SCRATCH,
)


def _combine_first_body(p_ref, o_ref):
    o_ref[...] = p_ref[0] + p_ref[1]


def _combine_rec_body(p_ref, xprev_ref, o_ref):
    o_ref[...] = 2.0 * (p_ref[0] + p_ref[1]) - xprev_ref[...]


def _combine_first(p):
    return pl.pallas_call(
        _combine_first_body,
        out_shape=jax.ShapeDtypeStruct((VP, F), jnp.float32),
    )(p)


def _combine_rec(p, xprev):
    return pl.pallas_call(
        _combine_rec_body,
        out_shape=jax.ShapeDtypeStruct((VP, F), jnp.float32),
    )(p, xprev)


_MM_ROWS = 2048


def _matmul_body(x0_ref, x1_ref, x2_ref, p3_ref, w_ref, b_ref, o_ref):
    x3 = 2.0 * (p3_ref[0] + p3_ref[1]) - x1_ref[...]
    acc = jnp.dot(x0_ref[...], w_ref[0], preferred_element_type=jnp.float32)
    acc += jnp.dot(x1_ref[...], w_ref[1], preferred_element_type=jnp.float32)
    acc += jnp.dot(x2_ref[...], w_ref[2], preferred_element_type=jnp.float32)
    acc += jnp.dot(x3, w_ref[3], preferred_element_type=jnp.float32)
    o_ref[...] = acc + b_ref[...]


def _matmul(x0, x1, x2, p3, weight, bias2d):
    xspec = pl.BlockSpec((_MM_ROWS, F), lambda i: (i, 0))
    return pl.pallas_call(
        _matmul_body,
        grid=(VP // _MM_ROWS,),
        in_specs=[xspec, xspec, xspec,
                  pl.BlockSpec((NC, _MM_ROWS, F), lambda i: (0, i, 0)),
                  pl.BlockSpec((KORD, F, F), lambda i: (0, 0, 0)),
                  pl.BlockSpec((1, F), lambda i: (0, 0))],
        out_specs=pl.BlockSpec((_MM_ROWS, F), lambda i: (i, 0)),
        out_shape=jax.ShapeDtypeStruct((VP, F), jnp.float32),
    )(x0, x1, x2, p3, weight, bias2d)


def kernel(lap_indices, lap_values, x, weight, bias):
    npad = EPAD - E
    spread = (jnp.arange(npad, dtype=jnp.int32) * 13) % V
    rows = jnp.concatenate(
        [lap_indices[0].astype(jnp.int32), spread]).reshape(NSEG_TOTAL, SEG)
    cols = jnp.concatenate(
        [lap_indices[1].astype(jnp.int32), spread]).reshape(NSEG_TOTAL, SEG)
    vals = jnp.concatenate(
        [lap_values.astype(jnp.float32), jnp.zeros((npad,), jnp.float32)])
    x0 = jnp.pad(x.reshape(V, F), ((0, VP - V), (0, 0)))
    bias2d = bias.reshape(1, F)
    weff = jnp.transpose(
        weight.reshape(KORD * F, F).reshape(F, KORD, F), (1, 0, 2))

    p = _spmm_sc(cols, rows, vals, x0)
    x1 = _combine_first(p)
    p = _spmm_sc(cols, rows, vals, x1)
    x2 = _combine_rec(p, x0)
    p3 = _spmm_sc(cols, rows, vals, x2)
    out = _matmul(x0, x1, x2, p3, weff, bias2d)
    return out[:V].reshape(1, V, F)


# speedup vs baseline: 10.9739x; 1.0905x over previous
import jax
import jax.numpy as jnp
from jax import lax
from jax.experimental import pallas as pl
from jax.experimental.pallas import tpu as pltpu
from jax.experimental.pallas import tpu_sc as plsc

V = 10000
VP = 10240
F = 128
E = 320000
KORD = 4
NC = 2
NS = 16
NW = NC * NS
L = 16
W = 128
SEG = 64
NBLK = 2560
EPAD = NBLK * W
NSEG_TOTAL = EPAD // SEG
WPT = NBLK // NW
SPT = WPT * 2
SEG_CHUNK = 32
PAIR_CHUNK = SEG_CHUNK // 2
NCHUNK = SPT // SEG_CHUNK
RPT = VP // NS

_mesh = plsc.VectorSubcoreMesh(core_axis_name="c", subcore_axis_name="s")

_SPMM_SCRATCH = [
    pltpu.VMEM((SEG_CHUNK, SEG), jnp.int32),
    pltpu.VMEM((SEG_CHUNK, SEG), jnp.int32),
    pltpu.VMEM((SEG_CHUNK * SEG,), jnp.float32),
    pltpu.VMEM((4 * SEG, F), jnp.float32),
    pltpu.VMEM_SHARED((VP, F), jnp.float32),
    pltpu.SemaphoreType.DMA,
    pltpu.SemaphoreType.DMA,
    pltpu.SemaphoreType.DMA,
    pltpu.SemaphoreType.DMA,
    pltpu.SemaphoreType.DMA,
    pltpu.SemaphoreType.DMA,
    pltpu.SemaphoreType.DMA,
    pltpu.SemaphoreType.DMA,
]


def _spmm_body(cols, rows, vals, xin, out,
               colw, roww, valw, ring, acc,
               gs0, gs1, gs2, gs3, ss0, ss1, ss2, ss3):
    gsems = (gs0, gs1, gs2, gs3)
    ssems = (ss0, ss1, ss2, ss3)
    cid = lax.axis_index("c")
    sid = lax.axis_index("s")
    wid = cid * NS + sid

    r0 = sid * RPT

    def _stage(c):
        pltpu.sync_copy(cols.at[pl.ds(wid * SPT + c * SEG_CHUNK, SEG_CHUNK)],
                        colw)
        pltpu.sync_copy(rows.at[pl.ds(wid * SPT + c * SEG_CHUNK, SEG_CHUNK)],
                        roww)
        pltpu.sync_copy(
            vals.at[pl.ds((wid * SPT + c * SEG_CHUNK) * SEG, SEG_CHUNK * SEG)],
            valw)

    def _gather(ls, k):
        pltpu.async_copy(xin.at[colw.at[ls]],
                         ring.at[pl.ds(k * SEG, SEG)], gsems[k])

    def _gwait(ls, k):
        pltpu.make_async_copy(xin.at[colw.at[ls]],
                              ring.at[pl.ds(k * SEG, SEG)], gsems[k]).wait()

    def _scale(ls, k):
        base = k * SEG

        def _s16(g, _):
            grp = valw[pl.ds(ls * SEG + g * L, L)]
            for u in range(L):
                e = base + g * L + u
                v = grp[u]
                for q in range(F // L):
                    ring[e, pl.ds(q * L, L)] = ring[e, pl.ds(q * L, L)] * v
            return 0

        lax.fori_loop(0, SEG // L, _s16, 0)

    def _scatter(ls, k):
        pltpu.async_copy(ring.at[pl.ds(k * SEG, SEG)],
                         acc.at[roww.at[ls]], ssems[k], add=True)

    def _swait(k):
        pltpu.make_async_copy(ring.at[pl.ds(k * SEG, SEG)],
                              acc.at[roww.at[0]], ssems[k]).wait()

    def _iter(i):
        for k in range(4):
            ls = 4 * i + k
            _gwait(ls, k)
            _scale(ls, k)
            _scatter(ls, k)
            j = (k + 3) % 4
            if k == 0:
                @pl.when(i > 0)
                def _():
                    _swait(j)

                _gather(ls + 3, j)
            else:
                _swait(j)
                _gather(ls + 3, j)

    _stage(0)
    _gather(0, 0)
    _gather(1, 1)
    _gather(2, 2)

    def _zero_rowbuf(e, _):
        z = jnp.zeros((L,), jnp.float32)
        for q in range(F // L):
            ring[3 * SEG + e, pl.ds(q * L, L)] = z
        return 0

    lax.fori_loop(0, SEG, _zero_rowbuf, 0)
    for k in range(RPT // SEG):
        pltpu.async_copy(ring.at[pl.ds(3 * SEG, SEG)],
                         acc.at[pl.ds(r0 + k * SEG, SEG)], ss0)
    for k in range(RPT // SEG):
        pltpu.make_async_copy(ring.at[pl.ds(3 * SEG, SEG)],
                              acc.at[pl.ds(r0 + k * SEG, SEG)], ss0).wait()
    plsc.subcore_barrier()

    def _body(i, _):
        _iter(i)
        return 0

    def _chunk(c, _):
        lax.fori_loop(0, SEG_CHUNK // 4 - 1, _body, 0)

        i = SEG_CHUNK // 4 - 1
        for k in range(4):
            ls = 4 * i + k
            _gwait(ls, k)
            _scale(ls, k)
            _scatter(ls, k)
            if k == 0:
                _swait(3)
                _gather(ls + 3, 3)

        for k in range(4):
            _swait(k)

        @pl.when(c + 1 < NCHUNK)
        def _():
            _stage(c + 1)
            _gather(0, 0)
            _gather(1, 1)
            _gather(2, 2)

        return 0

    lax.fori_loop(0, NCHUNK, _chunk, 0)

    plsc.subcore_barrier()

    for k in range(RPT // W):
        pltpu.async_copy(acc.at[pl.ds(r0 + k * W, W)],
                         out.at[cid, pl.ds(r0 + k * W, W)], ss0)
    for k in range(RPT // W):
        pltpu.make_async_copy(acc.at[pl.ds(r0 + k * W, W)],
                              out.at[cid, pl.ds(r0 + k * W, W)], ss0).wait()


_spmm_sc = pl.kernel(
    _spmm_body,
    out_type=jax.ShapeDtypeStruct((NC, VP, F), jnp.float32),
    mesh=_mesh,
    scratch_types=_SPMM_SCRATCH,
)


def _combine_first_body(p_ref, o_ref):
    o_ref[...] = p_ref[0] + p_ref[1]


def _combine_rec_body(p_ref, xprev_ref, o_ref):
    o_ref[...] = 2.0 * (p_ref[0] + p_ref[1]) - xprev_ref[...]


def _combine_first(p):
    return pl.pallas_call(
        _combine_first_body,
        out_shape=jax.ShapeDtypeStruct((VP, F), jnp.float32),
    )(p)


def _combine_rec(p, xprev):
    return pl.pallas_call(
        _combine_rec_body,
        out_shape=jax.ShapeDtypeStruct((VP, F), jnp.float32),
    )(p, xprev)


_MM_ROWS = 2048


def _matmul_body(x0_ref, x1_ref, x2_ref, p3_ref, w_ref, b_ref, o_ref):
    x3 = 2.0 * (p3_ref[0] + p3_ref[1]) - x1_ref[...]
    acc = jnp.dot(x0_ref[...], w_ref[0], preferred_element_type=jnp.float32)
    acc += jnp.dot(x1_ref[...], w_ref[1], preferred_element_type=jnp.float32)
    acc += jnp.dot(x2_ref[...], w_ref[2], preferred_element_type=jnp.float32)
    acc += jnp.dot(x3, w_ref[3], preferred_element_type=jnp.float32)
    o_ref[...] = acc + b_ref[...]


def _matmul(x0, x1, x2, p3, weight, bias2d):
    xspec = pl.BlockSpec((_MM_ROWS, F), lambda i: (i, 0))
    return pl.pallas_call(
        _matmul_body,
        grid=(VP // _MM_ROWS,),
        in_specs=[xspec, xspec, xspec,
                  pl.BlockSpec((NC, _MM_ROWS, F), lambda i: (0, i, 0)),
                  pl.BlockSpec((KORD, F, F), lambda i: (0, 0, 0)),
                  pl.BlockSpec((1, F), lambda i: (0, 0))],
        out_specs=pl.BlockSpec((_MM_ROWS, F), lambda i: (i, 0)),
        out_shape=jax.ShapeDtypeStruct((VP, F), jnp.float32),
    )(x0, x1, x2, p3, weight, bias2d)


def kernel(lap_indices, lap_values, x, weight, bias):
    npad = EPAD - E
    spread = (jnp.arange(npad, dtype=jnp.int32) * 13) % V
    rows = jnp.concatenate(
        [lap_indices[0].astype(jnp.int32), spread]).reshape(NSEG_TOTAL, SEG)
    cols = jnp.concatenate(
        [lap_indices[1].astype(jnp.int32), spread]).reshape(NSEG_TOTAL, SEG)
    vals = jnp.concatenate(
        [lap_values.astype(jnp.float32), jnp.zeros((npad,), jnp.float32)])
    x0 = jnp.pad(x.reshape(V, F), ((0, VP - V), (0, 0)))
    bias2d = bias.reshape(1, F)
    weff = jnp.transpose(
        weight.reshape(KORD * F, F).reshape(F, KORD, F), (1, 0, 2))

    p = _spmm_sc(cols, rows, vals, x0)
    x1 = _combine_first(p)
    p = _spmm_sc(cols, rows, vals, x1)
    x2 = _combine_rec(p, x0)
    p3 = _spmm_sc(cols, rows, vals, x2)
    out = _matmul(x0, x1, x2, p3, weff, bias2d)
    return out[:V].reshape(1, V, F)


# speedup vs baseline: 11.2035x; 1.0209x over previous
import jax
import jax.numpy as jnp
from jax import lax
from jax.experimental import pallas as pl
from jax.experimental.pallas import tpu as pltpu
from jax.experimental.pallas import tpu_sc as plsc

V = 10000
VP = 10240
F = 128
E = 320000
KORD = 4
NC = 2
NS = 16
NW = NC * NS
L = 16
W = 128
SEG = 64
NBLK = 2560
EPAD = NBLK * W
NSEG_TOTAL = EPAD // SEG
WPT = NBLK // NW
SPT = WPT * 2
SEG_CHUNK = 40
PAIR_CHUNK = SEG_CHUNK // 2
NCHUNK = SPT // SEG_CHUNK
RPT = VP // NS

_mesh = plsc.VectorSubcoreMesh(core_axis_name="c", subcore_axis_name="s")

_SPMM_SCRATCH = [
    pltpu.VMEM((SEG_CHUNK, SEG), jnp.int32),
    pltpu.VMEM((SEG_CHUNK, SEG), jnp.int32),
    pltpu.VMEM((SEG_CHUNK * SEG,), jnp.float32),
    pltpu.VMEM((4 * SEG, F), jnp.float32),
    pltpu.VMEM_SHARED((VP, F), jnp.float32),
    pltpu.SemaphoreType.DMA,
    pltpu.SemaphoreType.DMA,
    pltpu.SemaphoreType.DMA,
    pltpu.SemaphoreType.DMA,
    pltpu.SemaphoreType.DMA,
    pltpu.SemaphoreType.DMA,
    pltpu.SemaphoreType.DMA,
    pltpu.SemaphoreType.DMA,
]


def _spmm_body(cols, rows, vals, xin, out,
               colw, roww, valw, ring, acc,
               gs0, gs1, gs2, gs3, ss0, ss1, ss2, ss3):
    gsems = (gs0, gs1, gs2, gs3)
    ssems = (ss0, ss1, ss2, ss3)
    cid = lax.axis_index("c")
    sid = lax.axis_index("s")
    wid = cid * NS + sid

    r0 = sid * RPT

    def _stage(c):
        pltpu.sync_copy(cols.at[pl.ds(wid * SPT + c * SEG_CHUNK, SEG_CHUNK)],
                        colw)
        pltpu.sync_copy(rows.at[pl.ds(wid * SPT + c * SEG_CHUNK, SEG_CHUNK)],
                        roww)
        pltpu.sync_copy(
            vals.at[pl.ds((wid * SPT + c * SEG_CHUNK) * SEG, SEG_CHUNK * SEG)],
            valw)

    def _gather(ls, k):
        pltpu.async_copy(xin.at[colw.at[ls]],
                         ring.at[pl.ds(k * SEG, SEG)], gsems[k])

    def _gwait(ls, k):
        pltpu.make_async_copy(xin.at[colw.at[ls]],
                              ring.at[pl.ds(k * SEG, SEG)], gsems[k]).wait()

    def _scale(ls, k):
        base = k * SEG

        def _s16(g, _):
            grp = valw[pl.ds(ls * SEG + g * L, L)]
            for u in range(L):
                e = base + g * L + u
                v = grp[u]
                for q in range(F // L):
                    ring[e, pl.ds(q * L, L)] = ring[e, pl.ds(q * L, L)] * v
            return 0

        lax.fori_loop(0, SEG // L, _s16, 0)

    def _scatter(ls, k):
        pltpu.async_copy(ring.at[pl.ds(k * SEG, SEG)],
                         acc.at[roww.at[ls]], ssems[k], add=True)

    def _swait(k):
        pltpu.make_async_copy(ring.at[pl.ds(k * SEG, SEG)],
                              acc.at[roww.at[0]], ssems[k]).wait()

    def _iter(i):
        for k in range(4):
            ls = 4 * i + k
            _gwait(ls, k)
            _scale(ls, k)
            _scatter(ls, k)
            j = (k + 3) % 4
            if k == 0:
                @pl.when(i > 0)
                def _():
                    _swait(j)

                _gather(ls + 3, j)
            else:
                _swait(j)
                _gather(ls + 3, j)

    _stage(0)
    _gather(0, 0)
    _gather(1, 1)
    _gather(2, 2)

    def _zero_rowbuf(e, _):
        z = jnp.zeros((L,), jnp.float32)
        for q in range(F // L):
            ring[3 * SEG + e, pl.ds(q * L, L)] = z
        return 0

    lax.fori_loop(0, SEG, _zero_rowbuf, 0)
    for k in range(RPT // SEG):
        pltpu.async_copy(ring.at[pl.ds(3 * SEG, SEG)],
                         acc.at[pl.ds(r0 + k * SEG, SEG)], ss0)
    for k in range(RPT // SEG):
        pltpu.make_async_copy(ring.at[pl.ds(3 * SEG, SEG)],
                              acc.at[pl.ds(r0 + k * SEG, SEG)], ss0).wait()
    plsc.subcore_barrier()

    def _body(i, _):
        _iter(i)
        return 0

    def _chunk(c, _):
        lax.fori_loop(0, SEG_CHUNK // 4 - 1, _body, 0)

        i = SEG_CHUNK // 4 - 1
        for k in range(4):
            ls = 4 * i + k
            _gwait(ls, k)
            _scale(ls, k)
            _scatter(ls, k)
            if k == 0:
                _swait(3)
                _gather(ls + 3, 3)

        for k in range(4):
            _swait(k)

        @pl.when(c + 1 < NCHUNK)
        def _():
            _stage(c + 1)
            _gather(0, 0)
            _gather(1, 1)
            _gather(2, 2)

        return 0

    lax.fori_loop(0, NCHUNK, _chunk, 0)

    plsc.subcore_barrier()

    for k in range(RPT // W):
        pltpu.async_copy(acc.at[pl.ds(r0 + k * W, W)],
                         out.at[cid, pl.ds(r0 + k * W, W)], ss0)
    for k in range(RPT // W):
        pltpu.make_async_copy(acc.at[pl.ds(r0 + k * W, W)],
                              out.at[cid, pl.ds(r0 + k * W, W)], ss0).wait()


_spmm_sc = pl.kernel(
    _spmm_body,
    out_type=jax.ShapeDtypeStruct((NC, VP, F), jnp.float32),
    mesh=_mesh,
    scratch_types=_SPMM_SCRATCH,
)


def _combine_first_body(p_ref, o_ref):
    o_ref[...] = p_ref[0] + p_ref[1]


def _combine_rec_body(p_ref, xprev_ref, o_ref):
    o_ref[...] = 2.0 * (p_ref[0] + p_ref[1]) - xprev_ref[...]


def _combine_first(p):
    return pl.pallas_call(
        _combine_first_body,
        out_shape=jax.ShapeDtypeStruct((VP, F), jnp.float32),
    )(p)


def _combine_rec(p, xprev):
    return pl.pallas_call(
        _combine_rec_body,
        out_shape=jax.ShapeDtypeStruct((VP, F), jnp.float32),
    )(p, xprev)


_MM_ROWS = 2048


def _matmul_body(x0_ref, x1_ref, x2_ref, p3_ref, w_ref, b_ref, o_ref):
    x3 = 2.0 * (p3_ref[0] + p3_ref[1]) - x1_ref[...]
    acc = jnp.dot(x0_ref[...], w_ref[0], preferred_element_type=jnp.float32)
    acc += jnp.dot(x1_ref[...], w_ref[1], preferred_element_type=jnp.float32)
    acc += jnp.dot(x2_ref[...], w_ref[2], preferred_element_type=jnp.float32)
    acc += jnp.dot(x3, w_ref[3], preferred_element_type=jnp.float32)
    o_ref[...] = acc + b_ref[...]


def _matmul(x0, x1, x2, p3, weight, bias2d):
    xspec = pl.BlockSpec((_MM_ROWS, F), lambda i: (i, 0))
    return pl.pallas_call(
        _matmul_body,
        grid=(VP // _MM_ROWS,),
        in_specs=[xspec, xspec, xspec,
                  pl.BlockSpec((NC, _MM_ROWS, F), lambda i: (0, i, 0)),
                  pl.BlockSpec((KORD, F, F), lambda i: (0, 0, 0)),
                  pl.BlockSpec((1, F), lambda i: (0, 0))],
        out_specs=pl.BlockSpec((_MM_ROWS, F), lambda i: (i, 0)),
        out_shape=jax.ShapeDtypeStruct((VP, F), jnp.float32),
    )(x0, x1, x2, p3, weight, bias2d)


def kernel(lap_indices, lap_values, x, weight, bias):
    npad = EPAD - E
    spread = (jnp.arange(npad, dtype=jnp.int32) * 13) % V
    rows = jnp.concatenate(
        [lap_indices[0].astype(jnp.int32), spread]).reshape(NSEG_TOTAL, SEG)
    cols = jnp.concatenate(
        [lap_indices[1].astype(jnp.int32), spread]).reshape(NSEG_TOTAL, SEG)
    vals = jnp.concatenate(
        [lap_values.astype(jnp.float32), jnp.zeros((npad,), jnp.float32)])
    x0 = jnp.pad(x.reshape(V, F), ((0, VP - V), (0, 0)))
    bias2d = bias.reshape(1, F)
    weff = jnp.transpose(
        weight.reshape(KORD * F, F).reshape(F, KORD, F), (1, 0, 2))

    p = _spmm_sc(cols, rows, vals, x0)
    x1 = _combine_first(p)
    p = _spmm_sc(cols, rows, vals, x1)
    x2 = _combine_rec(p, x0)
    p3 = _spmm_sc(cols, rows, vals, x2)
    out = _matmul(x0, x1, x2, p3, weff, bias2d)
    return out[:V].reshape(1, V, F)
